# Initial kernel scaffold; baseline (speedup 1.0000x reference)
import jax
import jax.numpy as jnp
from jax.experimental import pallas as pl


def kernel(x, pos, edge_index, batch, emb, lin_W, mlp_W1, mlp_b1, mlp_W2, mlp_b2, vW1, vb1, vW2, vb2, uW1, ub1, uW2, ub2, tW1, tb1, tW2, tb2, bn_g, bn_b):
    raise NotImplementedError("write your pallas kernel here")


# speedup vs baseline: 3.0116x; 3.0116x over previous
import functools
import numpy as np
import jax
import jax.numpy as jnp
from jax import lax
from jax.experimental import pallas as pl
from jax.experimental.pallas import tpu as pltpu
from jax.experimental.pallas import tpu_sc as plsc

N_NODES = 10000
N_EDGES = 320000
N_GRAPHS = 512
HIDDEN = 128
FILTERS = 128
NG = 50
OUT_CH = 32
N_LAYERS = 6
CUTOFF = 10.0

NC = 2
NS = 16
NW = NC * NS
EPT = N_EDGES // NW
CHUNK = 80
NCHUNK = EPT // CHUNK
ZROWS = 125
RPT = N_NODES // NS

FB = 8000
FBLOCKS = N_EDGES // FB
NB = 2000
NBLOCKS = N_NODES // NB

_LOG2 = float(np.log(2.0))
_GSTEP = CUTOFF / (NG - 1)
_COEFF = float(-0.5 / _GSTEP ** 2)
_BN_SCALE = float(1.0 / np.sqrt(1.0 + 1e-5))


def _offsets64():
    gi = lax.broadcasted_iota(jnp.int32, (1, 64), 1)
    return jnp.where(gi < NG, gi.astype(jnp.float32) * _GSTEP, 1.0e9)


def _ssp(x):
    return jnp.maximum(x, 0.0) + jnp.log1p(jnp.exp(-jnp.abs(x))) - _LOG2


def _dot(a, b, precision=lax.Precision.HIGHEST):
    return lax.dot_general(a, b, (((1,), (0,)), ((), ())),
                           preferred_element_type=jnp.float32,
                           precision=precision)


def _dott(a, b):
    return lax.dot_general(a, b, (((0,), (0,)), ((), ())),
                           preferred_element_type=jnp.float32,
                           precision=lax.Precision.HIGHEST)


def _rows(row, n):
    ones = jnp.ones((1, n), jnp.float32)
    return _dott(row, ones)


def _dist_body(px_h, py_h, pz_h, row_h, col_h, out_h,
               px, py, pz, ridx, cidx, obuf):
    c = lax.axis_index("c")
    s = lax.axis_index("s")
    wid = c * NS + s
    pltpu.sync_copy(px_h, px)
    pltpu.sync_copy(py_h, py)
    pltpu.sync_copy(pz_h, pz)

    def body(g, carry):
        base = wid * EPT + g * CHUNK
        pltpu.sync_copy(row_h.at[pl.ds(base, CHUNK)], ridx)
        pltpu.sync_copy(col_h.at[pl.ds(base, CHUNK)], cidx)
        for i in range(CHUNK // 16):
            sl = pl.ds(i * 16, 16)
            r = ridx[sl]
            cc = cidx[sl]
            dx = plsc.load_gather(px, [r]) - plsc.load_gather(px, [cc])
            dy = plsc.load_gather(py, [r]) - plsc.load_gather(py, [cc])
            dz = plsc.load_gather(pz, [r]) - plsc.load_gather(pz, [cc])
            obuf[sl] = dx * dx + dy * dy + dz * dz
        pltpu.sync_copy(obuf, out_h.at[pl.ds(base, CHUNK)])
        return carry

    lax.fori_loop(0, NCHUNK, body, 0)


_SC_PARAMS = pltpu.CompilerParams(
    use_tc_tiling_on_sc=False, needs_layout_passes=False)

_dist_call = pl.kernel(
    _dist_body,
    out_type=jax.ShapeDtypeStruct((N_EDGES,), jnp.float32),
    mesh=plsc.VectorSubcoreMesh(core_axis_name="c", subcore_axis_name="s", num_cores=NC, num_subcores=NS),
    compiler_params=_SC_PARAMS,
    scratch_types=[
        pltpu.VMEM((N_NODES,), jnp.float32),
        pltpu.VMEM((N_NODES,), jnp.float32),
        pltpu.VMEM((N_NODES,), jnp.float32),
        pltpu.VMEM((CHUNK,), jnp.int32),
        pltpu.VMEM((CHUNK,), jnp.int32),
        pltpu.VMEM((CHUNK,), jnp.float32),
    ],
)


def _msg_body(vp_h, w_h, row_h, col_h, zeros_h, out_h, accum,
              ridxA, ridxB, cidxA, cidxB, scidxA, scidxB,
              vjA, vjB, wbufA, wbufB,
              gsemA, gsemB, wsemA, wsemB, ssemA, ssemB):
    c = lax.axis_index("c")
    s = lax.axis_index("s")
    wid = c * NS + s
    tbase = wid * EPT

    pltpu.sync_copy(zeros_h.at[pl.ds(s * RPT, RPT)],
                    accum.at[pl.ds(s * RPT, RPT)])
    plsc.subcore_barrier()

    def fetch(g, ridx, cidx, vj, wbuf, gsem, wsem, ssem, wait_sc):
        base = tbase + g * CHUNK
        pltpu.sync_copy(row_h.at[pl.ds(base, CHUNK)], ridx)
        pltpu.sync_copy(col_h.at[pl.ds(base, CHUNK)], cidx)
        pltpu.async_copy(vp_h.at[ridx], vj, gsem)
        if wait_sc:
            pltpu.make_async_copy(wbuf, accum.at[cidx], ssem).wait()
        pltpu.async_copy(w_h.at[pl.ds(base, CHUNK)], wbuf, wsem)

    def consume(ridx, cidx, scidx, vj, wbuf, gsem, wsem, ssem):
        pltpu.make_async_copy(vp_h.at[ridx], vj, gsem).wait()
        pltpu.make_async_copy(w_h.at[pl.ds(0, CHUNK)], wbuf, wsem).wait()
        for t in range(CHUNK // 16):
            sl = pl.ds(t * 16, 16)
            scidx[sl] = cidx[sl]

        @plsc.parallel_loop(0, CHUNK, unroll=4)
        def _mul(i):
            for j in range(FILTERS // 16):
                sl = pl.ds(j * 16, 16)
                wbuf[i, sl] = wbuf[i, sl] * vj[i, sl]

        pltpu.async_copy(wbuf, accum.at[scidx], ssem, add=True)

    fetch(0, ridxA, cidxA, vjA, wbufA, gsemA, wsemA, ssemA, False)
    fetch(1, ridxB, cidxB, vjB, wbufB, gsemB, wsemB, ssemB, False)
    consume(ridxA, cidxA, scidxA, vjA, wbufA, gsemA, wsemA, ssemA)
    fetch(2, ridxA, cidxA, vjA, wbufA, gsemA, wsemA, ssemA, True)
    consume(ridxB, cidxB, scidxB, vjB, wbufB, gsemB, wsemB, ssemB)

    def body(k, carry):
        fetch(2 * k + 1, ridxB, cidxB, vjB, wbufB, gsemB, wsemB, ssemB, True)
        consume(ridxA, cidxA, scidxA, vjA, wbufA, gsemA, wsemA, ssemA)
        fetch(2 * k + 2, ridxA, cidxA, vjA, wbufA, gsemA, wsemA, ssemA, True)
        consume(ridxB, cidxB, scidxB, vjB, wbufB, gsemB, wsemB, ssemB)
        return carry

    lax.fori_loop(1, (NCHUNK - 1) // 2, body, 0)
    consume(ridxA, cidxA, scidxA, vjA, wbufA, gsemA, wsemA, ssemA)
    pltpu.make_async_copy(wbufB, accum.at[scidxB], ssemB).wait()
    pltpu.make_async_copy(wbufA, accum.at[scidxA], ssemA).wait()
    plsc.subcore_barrier()
    for k in range(RPT // ZROWS):
        r0 = s * RPT + k * ZROWS
        pltpu.sync_copy(accum.at[pl.ds(r0, ZROWS)],
                        out_h.at[pl.ds(c * N_NODES + r0, ZROWS)])


_msg_call = pl.kernel(
    _msg_body,
    out_type=jax.ShapeDtypeStruct((NC * N_NODES, FILTERS), jnp.float32),
    mesh=plsc.VectorSubcoreMesh(core_axis_name="c", subcore_axis_name="s", num_cores=NC, num_subcores=NS),
    compiler_params=_SC_PARAMS,
    scratch_types=(
        [pltpu.VMEM_SHARED((N_NODES, FILTERS), jnp.float32)]
        + [pltpu.VMEM((CHUNK,), jnp.int32)] * 6
        + [pltpu.VMEM((CHUNK, FILTERS), jnp.float32)] * 4
        + [pltpu.SemaphoreType.DMA] * 6
    ),
)


def _expand_body(d2_ref, de_ref, c_ref):
    d2 = d2_ref[0]
    dist = jnp.sqrt(d2 + 1e-12)
    db64 = _rows(dist, 64)
    de_ref[...] = jnp.exp(_COEFF * (db64 - _offsets64()) ** 2)
    c_ref[0] = 0.5 * (jnp.cos(dist * (np.pi / CUTOFF)) + 1.0)


def _expand_call(d2r3):
    return pl.pallas_call(
        _expand_body,
        grid=(FBLOCKS,),
        in_specs=[pl.BlockSpec((1, 1, FB), lambda i: (i, 0, 0))],
        out_specs=[
            pl.BlockSpec((FB, 64), lambda i: (i, 0)),
            pl.BlockSpec((1, 1, FB), lambda i: (i, 0, 0)),
        ],
        out_shape=[
            jax.ShapeDtypeStruct((N_EDGES, 64), jnp.float32),
            jax.ShapeDtypeStruct((FBLOCKS, 1, FB), jnp.float32),
        ],
    )(d2r3)


def _filter_body(de_ref, c_ref, w1_ref, b1_ref, w2_ref, b2_ref, out_ref):
    h = _ssp(_dot(de_ref[...], w1_ref[...])
             + b1_ref[...])
    h = _dot(h, w2_ref[...]) + b2_ref[...]
    out_ref[...] = h * _rows(c_ref[0], FILTERS)


def _filter_call(de, crow3, w1p, b1, w2, b2):
    return pl.pallas_call(
        _filter_body,
        grid=(FBLOCKS,),
        in_specs=[
            pl.BlockSpec((FB, 64), lambda i: (i, 0)),
            pl.BlockSpec((1, 1, FB), lambda i: (i, 0, 0)),
            pl.BlockSpec((64, FILTERS), lambda i: (0, 0)),
            pl.BlockSpec((1, FILTERS), lambda i: (0, 0)),
            pl.BlockSpec((FILTERS, FILTERS), lambda i: (0, 0)),
            pl.BlockSpec((1, FILTERS), lambda i: (0, 0)),
        ],
        out_specs=pl.BlockSpec((FB, FILTERS), lambda i: (i, 0)),
        out_shape=jax.ShapeDtypeStruct((N_EDGES, FILTERS), jnp.float32),
    )(de, crow3, w1p, b1, w2, b2)


def _prologue_body(zf_ref, emb_ref, lin_ref, v_ref, vp_ref):
    zb = _rows(zf_ref[0], HIDDEN)
    ids = lax.broadcasted_iota(jnp.int32, (1, HIDDEN), 1).astype(jnp.float32)
    oneh = (jnp.abs(zb - ids) < 0.5).astype(jnp.float32)
    v = _dot(oneh, emb_ref[...])
    v_ref[...] = v
    vp_ref[...] = _dot(v, lin_ref[...])


def _prologue_call(zf3, embp, lin0):
    return pl.pallas_call(
        _prologue_body,
        grid=(NBLOCKS,),
        in_specs=[
            pl.BlockSpec((1, 1, NB), lambda i: (i, 0, 0)),
            pl.BlockSpec((HIDDEN, HIDDEN), lambda i: (0, 0)),
            pl.BlockSpec((HIDDEN, FILTERS), lambda i: (0, 0)),
        ],
        out_specs=[
            pl.BlockSpec((NB, HIDDEN), lambda i: (i, 0)),
            pl.BlockSpec((NB, FILTERS), lambda i: (i, 0)),
        ],
        out_shape=[
            jax.ShapeDtypeStruct((N_NODES, HIDDEN), jnp.float32),
            jax.ShapeDtypeStruct((N_NODES, FILTERS), jnp.float32),
        ],
    )(zf3, embp, lin0)


def _node_body(parts_ref, v_ref, w1_ref, b1_ref, w2_ref, b2_ref, lin_ref,
               vo_ref, vp_ref):
    agg = parts_ref[0] + parts_ref[1]
    h = _ssp(_dot(agg, w1_ref[...]) + b1_ref[...])
    o = _dot(h, w2_ref[...]) + b2_ref[...]
    vn = v_ref[...] + o
    vo_ref[...] = vn
    vp_ref[...] = _dot(vn, lin_ref[...])


def _node_call(parts3, v, w1, b1, w2, b2, lin_next):
    return pl.pallas_call(
        _node_body,
        grid=(NBLOCKS,),
        in_specs=[
            pl.BlockSpec((NC, NB, FILTERS), lambda i: (0, i, 0)),
            pl.BlockSpec((NB, HIDDEN), lambda i: (i, 0)),
            pl.BlockSpec((FILTERS, HIDDEN), lambda i: (0, 0)),
            pl.BlockSpec((1, HIDDEN), lambda i: (0, 0)),
            pl.BlockSpec((HIDDEN, HIDDEN), lambda i: (0, 0)),
            pl.BlockSpec((1, HIDDEN), lambda i: (0, 0)),
            pl.BlockSpec((HIDDEN, FILTERS), lambda i: (0, 0)),
        ],
        out_specs=[
            pl.BlockSpec((NB, HIDDEN), lambda i: (i, 0)),
            pl.BlockSpec((NB, FILTERS), lambda i: (i, 0)),
        ],
        out_shape=[
            jax.ShapeDtypeStruct((N_NODES, HIDDEN), jnp.float32),
            jax.ShapeDtypeStruct((N_NODES, FILTERS), jnp.float32),
        ],
    )(parts3, v, w1, b1, w2, b2, lin_next)


def _readout_body(v_ref, bf_ref, uw1_ref, ub1_ref, uw2_ref, ub2_ref,
                  tw1_ref, tb1_ref, tw2_ref, tb2_ref, bng_ref, bnb_ref,
                  out_ref, uacc):
    i = pl.program_id(0)

    @pl.when(i == 0)
    def _init():
        uacc[...] = jnp.zeros_like(uacc)

    h = _ssp(_dot(v_ref[...], uw1_ref[...]) + ub1_ref[...])
    h = _dot(h, uw2_ref[...]) + ub2_ref[...]
    bb = _rows(bf_ref[0], N_GRAPHS)
    gid = lax.broadcasted_iota(jnp.int32, (1, N_GRAPHS), 1).astype(jnp.float32)
    oneh = (jnp.abs(bb - gid) < 0.5).astype(jnp.float32)
    uacc[...] += _dott(oneh, h)

    @pl.when(i == NBLOCKS - 1)
    def _head():
        u = uacc[...]
        t1 = jnp.maximum(_dot(u, tw1_ref[...]) + tb1_ref[...], 0.0)
        g = t1 * (bng_ref[...] * _BN_SCALE) + bnb_ref[...]
        out_ref[...] = _dot(g, tw2_ref[...]) + tb2_ref[...]


def _readout_call(v, bf3, uw1, ub1, uw2p, ub2p, tw1p, tb1, tw2p, tb2p,
                  bng, bnb):
    return pl.pallas_call(
        _readout_body,
        grid=(NBLOCKS,),
        in_specs=[
            pl.BlockSpec((NB, HIDDEN), lambda i: (i, 0)),
            pl.BlockSpec((1, 1, NB), lambda i: (i, 0, 0)),
            pl.BlockSpec((HIDDEN, 64), lambda i: (0, 0)),
            pl.BlockSpec((1, 64), lambda i: (0, 0)),
            pl.BlockSpec((64, HIDDEN), lambda i: (0, 0)),
            pl.BlockSpec((1, HIDDEN), lambda i: (0, 0)),
            pl.BlockSpec((HIDDEN, 64), lambda i: (0, 0)),
            pl.BlockSpec((1, 64), lambda i: (0, 0)),
            pl.BlockSpec((64, HIDDEN), lambda i: (0, 0)),
            pl.BlockSpec((1, HIDDEN), lambda i: (0, 0)),
            pl.BlockSpec((1, 64), lambda i: (0, 0)),
            pl.BlockSpec((1, 64), lambda i: (0, 0)),
        ],
        out_specs=pl.BlockSpec((N_GRAPHS, HIDDEN), lambda i: (0, 0)),
        out_shape=jax.ShapeDtypeStruct((N_GRAPHS, HIDDEN), jnp.float32),
        scratch_shapes=[pltpu.VMEM((N_GRAPHS, HIDDEN), jnp.float32)],
    )(v, bf3, uw1, ub1, uw2p, ub2p, tw1p, tb1, tw2p, tb2p, bng, bnb)


def kernel(x, pos, edge_index, batch, emb, lin_W, mlp_W1, mlp_b1, mlp_W2,
           mlp_b2, vW1, vb1, vW2, vb2, uW1, ub1, uW2, ub2, tW1, tb1, tW2,
           tb2, bn_g, bn_b):
    f32 = jnp.float32
    row = edge_index[0].astype(jnp.int32)
    col = edge_index[1].astype(jnp.int32)
    posT = pos.T.astype(f32)
    zf3 = x.reshape(-1).astype(f32).reshape(NBLOCKS, 1, NB)
    bf3 = batch.astype(f32).reshape(NBLOCKS, 1, NB)

    embp = jnp.zeros((HIDDEN, HIDDEN), f32).at[:100].set(emb)
    w1p = jnp.zeros((N_LAYERS, 64, FILTERS), f32).at[:, :NG].set(mlp_W1)
    b1r = mlp_b1.reshape(N_LAYERS, 1, FILTERS)
    b2r = mlp_b2.reshape(N_LAYERS, 1, FILTERS)
    vb1r = vb1.reshape(N_LAYERS, 1, HIDDEN)
    vb2r = vb2.reshape(N_LAYERS, 1, HIDDEN)
    ub1r = ub1.reshape(1, 64)
    uw2p = jnp.zeros((64, HIDDEN), f32).at[:, :OUT_CH].set(uW2)
    ub2p = jnp.zeros((1, HIDDEN), f32).at[0, :OUT_CH].set(ub2)
    tw1p = jnp.zeros((HIDDEN, 64), f32).at[:OUT_CH].set(tW1)
    tb1r = tb1.reshape(1, 64)
    tw2p = jnp.zeros((64, HIDDEN), f32).at[:, :1].set(tW2)
    tb2p = jnp.zeros((1, HIDDEN), f32).at[0, :1].set(tb2)
    bngr = bn_g.reshape(1, 64)
    bnbr = bn_b.reshape(1, 64)

    zeros_nf = jnp.zeros((N_NODES, FILTERS), f32)
    d2 = _dist_call(posT[0], posT[1], posT[2], row, col)
    d2r3 = d2.reshape(FBLOCKS, 1, FB)
    de, crow3 = _expand_call(d2r3)

    v, vp = _prologue_call(zf3, embp, lin_W[0])
    w_e = _filter_call(de, crow3, w1p[0], b1r[0], mlp_W2[0], b2r[0])
    for l in range(N_LAYERS):
        parts = _msg_call(vp, w_e, row, col, zeros_nf)
        if l + 1 < N_LAYERS:
            w_e = _filter_call(de, crow3, w1p[l + 1], b1r[l + 1],
                               mlp_W2[l + 1], b2r[l + 1])
        parts3 = parts.reshape(NC, N_NODES, FILTERS)
        lin_next = lin_W[(l + 1) % N_LAYERS]
        v, vp = _node_call(parts3, v, vW1[l], vb1r[l], vW2[l], vb2r[l],
                           lin_next)

    out = _readout_call(v, bf3, uW1, ub1r, uw2p, ub2p, tw1p, tb1r, tw2p,
                        tb2p, bngr, bnbr)
    return out[:, :1]
